# speedup vs baseline: 8.4194x; 8.4194x over previous
import functools

import jax
import jax.numpy as jnp
from jax import lax
from jax.experimental import pallas as pl
from jax.experimental.pallas import tpu as pltpu
from jax.experimental.pallas import tpu_sc as plsc

_ROWS = 128
_LANES = 16


def _sc_gather_fn(B, DEG, F, NC, NS):
    NW = NC * NS
    BPW = B // NW
    NPC = _ROWS // DEG
    NCHUNK = (BPW * DEG) // _ROWS
    NF = F // _LANES
    SELF_CHUNKS = BPW // _ROWS

    mesh = plsc.VectorSubcoreMesh(core_axis_name="c", subcore_axis_name="s")

    @functools.partial(
        pl.kernel,
        out_type=jax.ShapeDtypeStruct((2, B, F), jnp.float32),
        mesh=mesh,
        scratch_types=[
            pltpu.VMEM((SELF_CHUNKS, _ROWS), jnp.int32),
            pltpu.VMEM((NCHUNK, _ROWS), jnp.int32),
            pltpu.VMEM((_ROWS, F), jnp.float32),
            pltpu.VMEM((_ROWS, F), jnp.float32),
            pltpu.VMEM((BPW, F), jnp.float32),
            pltpu.SemaphoreType.DMA,
            pltpu.SemaphoreType.DMA,
        ],
    )
    def k(feat_hbm, ni_hbm, bn_hbm, out_hbm, bn_v, ni_v, buf0, buf1, acc_v,
          sem0, sem1):
        wid = lax.axis_index("s") * NC + lax.axis_index("c")
        base = wid * BPW
        bufs = (buf0, buf1)
        sems = (sem0, sem1)

        pltpu.sync_copy(bn_hbm.at[pl.ds(wid * SELF_CHUNKS, SELF_CHUNKS)], bn_v)
        pltpu.sync_copy(ni_hbm.at[pl.ds(wid * NCHUNK, NCHUNK)], ni_v)

        for c in range(SELF_CHUNKS):
            pltpu.async_copy(feat_hbm.at[bn_v.at[c]], buf0, sem0).wait()
            pltpu.sync_copy(buf0, out_hbm.at[0, pl.ds(base + c * _ROWS, _ROWS)])

        for b in range(2):
            pltpu.async_copy(feat_hbm.at[ni_v.at[b]], bufs[b], sems[b])

        @pl.loop(0, NCHUNK, step=2)
        def _(g):
            for b in range(2):
                chunk = g + b
                buf = bufs[b]
                pltpu.make_async_copy(feat_hbm.at[ni_v.at[chunk]], buf,
                                      sems[b]).wait()
                for j in range(NPC):
                    rb = j * DEG

                    def rbody(r, accs, rb=rb, buf=buf):
                        return tuple(
                            accs[f] + buf[rb + r, pl.ds(f * _LANES, _LANES)]
                            for f in range(NF))

                    accs = lax.fori_loop(
                        0, DEG, rbody,
                        tuple(jnp.zeros((_LANES,), jnp.float32)
                              for _ in range(NF)))
                    node = chunk * NPC + j
                    for f in range(NF):
                        acc_v[node, pl.ds(f * _LANES, _LANES)] = accs[f]

                @pl.when(chunk + 2 < NCHUNK)
                def _(buf=buf, b=b, chunk=chunk):
                    pltpu.async_copy(feat_hbm.at[ni_v.at[chunk + 2]], buf,
                                     sems[b])

        pltpu.sync_copy(acc_v, out_hbm.at[1, pl.ds(base, BPW)])

    return k


def _tc_head_fn(B, DEG, F, H, C, BLK):
    inv_deg = 1.0 / DEG

    def body(s_ref, n_ref, w1_ref, w2_ref, wc_ref, o_ref):
        x = jnp.dot(s_ref[0], w1_ref[...], preferred_element_type=jnp.float32)
        x = x + jnp.dot(n_ref[0], w2_ref[...] * inv_deg,
                        preferred_element_type=jnp.float32)
        h = jnp.maximum(x, 0.0)
        o_ref[...] = jnp.dot(h, wc_ref[...], preferred_element_type=jnp.float32)

    return pl.pallas_call(
        body,
        grid=(B // BLK,),
        in_specs=[
            pl.BlockSpec((1, BLK, F), lambda i: (0, i, 0)),
            pl.BlockSpec((1, BLK, F), lambda i: (1, i, 0)),
            pl.BlockSpec((F, H), lambda i: (0, 0)),
            pl.BlockSpec((F, H), lambda i: (0, 0)),
            pl.BlockSpec((H, C), lambda i: (0, 0)),
        ],
        out_specs=pl.BlockSpec((BLK, C), lambda i: (i, 0)),
        out_shape=jax.ShapeDtypeStruct((B, C), jnp.float32),
    )


def kernel(features, neigh_idx, batch_nodes, W_enc, W_cls):
    B, DEG = neigh_idx.shape
    F = features.shape[1]
    H = W_enc.shape[1]
    C = W_cls.shape[1]

    info = plsc.get_sparse_core_info()
    NC, NS = info.num_cores, info.num_subcores

    ni = neigh_idx.astype(jnp.int32).reshape(B * DEG // _ROWS, _ROWS)
    bn = batch_nodes.astype(jnp.int32).reshape(B // _ROWS, _ROWS)

    combined = _sc_gather_fn(B, DEG, F, NC, NS)(features, ni, bn)
    scores = _tc_head_fn(B, DEG, F, H, C, BLK=512)(
        combined, combined, W_enc[:F], W_enc[F:], W_cls)
    return scores


# speedup vs baseline: 8.4650x; 1.0054x over previous
import functools

import jax
import jax.numpy as jnp
from jax import lax
from jax.experimental import pallas as pl
from jax.experimental.pallas import tpu as pltpu
from jax.experimental.pallas import tpu_sc as plsc

_ROWS = 128
_LANES = 16


def _sc_gather_fn(B, DEG, F, NC, NS):
    NW = NC * NS
    BPW = B // NW
    NPC = _ROWS // DEG
    NCHUNK = (BPW * DEG) // _ROWS
    NF = F // _LANES
    SELF_CHUNKS = BPW // _ROWS

    mesh = plsc.VectorSubcoreMesh(core_axis_name="c", subcore_axis_name="s")

    @functools.partial(
        pl.kernel,
        out_type=(jax.ShapeDtypeStruct((B, F), jnp.float32),
                  jax.ShapeDtypeStruct((B, F), jnp.float32)),
        mesh=mesh,
        scratch_types=[
            pltpu.VMEM((SELF_CHUNKS, _ROWS), jnp.int32),
            pltpu.VMEM((NCHUNK, _ROWS), jnp.int32),
            pltpu.VMEM((_ROWS, F), jnp.float32),
            pltpu.VMEM((_ROWS, F), jnp.float32),
            pltpu.VMEM((_ROWS, F), jnp.float32),
            pltpu.VMEM((_ROWS, F), jnp.float32),
            pltpu.VMEM((BPW, F), jnp.float32),
            pltpu.SemaphoreType.DMA,
            pltpu.SemaphoreType.DMA,
            pltpu.SemaphoreType.DMA,
            pltpu.SemaphoreType.DMA,
        ],
    )
    def k(feat_hbm, ni_hbm, bn_hbm, self_hbm, nsum_hbm, bn_v, ni_v,
          buf0, buf1, buf2, buf3, acc_v, sem0, sem1, sem2, sem3):
        wid = lax.axis_index("s") * NC + lax.axis_index("c")
        base = wid * BPW
        bufs = (buf0, buf1, buf2, buf3)
        sems = (sem0, sem1, sem2, sem3)

        pltpu.sync_copy(bn_hbm.at[pl.ds(wid * SELF_CHUNKS, SELF_CHUNKS)], bn_v)
        pltpu.sync_copy(ni_hbm.at[pl.ds(wid * NCHUNK, NCHUNK)], ni_v)

        for c in range(SELF_CHUNKS):
            pltpu.async_copy(feat_hbm.at[bn_v.at[c]], bufs[c], sems[c])
        for n in range(2):
            pltpu.async_copy(feat_hbm.at[ni_v.at[n]], bufs[2 + n], sems[2 + n])

        for c in range(SELF_CHUNKS):
            pltpu.make_async_copy(feat_hbm.at[bn_v.at[c]], bufs[c],
                                  sems[c]).wait()
            pltpu.sync_copy(bufs[c], self_hbm.at[pl.ds(base + c * _ROWS,
                                                       _ROWS)])
        for n in range(2, 4):
            pltpu.async_copy(feat_hbm.at[ni_v.at[n]], bufs[n - 2],
                             sems[n - 2])

        @pl.loop(0, NCHUNK, step=4)
        def _(g):
            for b in range(4):
                chunk = g + b
                buf = bufs[(b + 2) % 4]
                sem = sems[(b + 2) % 4]
                pltpu.make_async_copy(feat_hbm.at[ni_v.at[chunk]], buf,
                                      sem).wait()
                for j in range(NPC):
                    rb = j * DEG

                    @pl.loop(
                        0, DEG,
                        init_carry=tuple(
                            jnp.zeros((_LANES,), jnp.float32)
                            for _ in range(NF)),
                        unroll=8)
                    def accs(r, carry, rb=rb, buf=buf):
                        return tuple(
                            carry[f] + buf[rb + r, pl.ds(f * _LANES, _LANES)]
                            for f in range(NF))

                    node = chunk * NPC + j
                    for f in range(NF):
                        acc_v[node, pl.ds(f * _LANES, _LANES)] = accs[f]

                @pl.when(chunk + 4 < NCHUNK)
                def _(buf=buf, sem=sem, chunk=chunk):
                    pltpu.async_copy(feat_hbm.at[ni_v.at[chunk + 4]], buf, sem)

        pltpu.sync_copy(acc_v, nsum_hbm.at[pl.ds(base, BPW)])

    return k


def _tc_head_fn(B, DEG, F, H, C, BLK):
    inv_deg = 1.0 / DEG

    def body(s_ref, n_ref, w1_ref, w2_ref, wc_ref, o_ref):
        x = jnp.dot(s_ref[...], w1_ref[...],
                    preferred_element_type=jnp.float32)
        x = x + jnp.dot(n_ref[...] * inv_deg, w2_ref[...],
                        preferred_element_type=jnp.float32)
        h = jnp.maximum(x, 0.0)
        o_ref[...] = jnp.dot(h, wc_ref[...], preferred_element_type=jnp.float32)

    return pl.pallas_call(
        body,
        grid=(B // BLK,),
        in_specs=[
            pl.BlockSpec((BLK, F), lambda i: (i, 0)),
            pl.BlockSpec((BLK, F), lambda i: (i, 0)),
            pl.BlockSpec((F, H), lambda i: (0, 0)),
            pl.BlockSpec((F, H), lambda i: (0, 0)),
            pl.BlockSpec((H, C), lambda i: (0, 0)),
        ],
        out_specs=pl.BlockSpec((BLK, C), lambda i: (i, 0)),
        out_shape=jax.ShapeDtypeStruct((B, C), jnp.float32),
    )


def kernel(features, neigh_idx, batch_nodes, W_enc, W_cls):
    B, DEG = neigh_idx.shape
    N, F = features.shape
    H = W_enc.shape[1]
    C = W_cls.shape[1]

    info = plsc.get_sparse_core_info()
    NC, NS = info.num_cores, info.num_subcores

    ni = neigh_idx.astype(jnp.int32).reshape(B * DEG // _ROWS, _ROWS)
    bn = batch_nodes.astype(jnp.int32).reshape(B // _ROWS, _ROWS)

    self32, nsum32 = _sc_gather_fn(B, DEG, F, NC, NS)(features, ni, bn)
    scores = _tc_head_fn(B, DEG, F, H, C, BLK=512)(
        self32, nsum32, W_enc[:F], W_enc[F:], W_cls)
    return scores
